# baseline (speedup 1.0000x reference)
import functools

import jax
import jax.numpy as jnp
from jax import lax
from jax.experimental import pallas as pl
from jax.experimental.pallas import tpu as pltpu
from jax.experimental.pallas import tpu_sc as plsc

_N = 1000000
_D = 32
_K = 16
_S = 1000
_SPAD = 1024
_CT = (_N + 127) // 128
_LCOLS = (_CT + 15) // 16
_CAP = 32
_SENTINEL = 0x0FFFFFFF
_EPS2 = 1e-16


def _rsqrt(x):
    xi = plsc.bitcast(x, jnp.int32)
    yi = jnp.int32(0x5F3759DF) - lax.shift_right_arithmetic(xi, 1)
    y = plsc.bitcast(yi, jnp.float32)
    for _ in range(3):
        y = y * (1.5 - 0.5 * x * y * y)
    return y


def _splat(x):
    return jnp.full((16,), x, jnp.int32)


def _make_sc_kernel():
    mesh = plsc.VectorSubcoreMesh(core_axis_name="c", subcore_axis_name="s")

    @functools.partial(
        pl.kernel,
        out_type=jax.ShapeDtypeStruct((32, 16), jnp.float32),
        mesh=mesh,
        compiler_params=pltpu.CompilerParams(
            needs_layout_passes=False, use_tc_tiling_on_sc=True),
        scratch_types=[
            pltpu.VMEM((12, 32, 128), jnp.float32),
            pltpu.VMEM((8, 16, 128), jnp.int32),
            pltpu.VMEM((8, 128), jnp.float32),
            pltpu.VMEM((4, 128), jnp.int32),
            pltpu.VMEM((64, 128), jnp.int32),
            pltpu.VMEM((128, 128), jnp.float32),
            pltpu.VMEM((_LCOLS * _CAP + 32,), jnp.int32),
            pltpu.VMEM((512,), jnp.int32),
            pltpu.VMEM((16,), jnp.float32),
            pltpu.VMEM((32,), jnp.int32),
            pltpu.SMEM((512,), jnp.int32),
            pltpu.VMEM_SHARED((64, 128), jnp.int32),
            pltpu.VMEM_SHARED((128, 128), jnp.float32),
            pltpu.SemaphoreType.DMA,
            pltpu.SemaphoreType.DMA,
            pltpu.SemaphoreType.DMA,
        ],
    )
    def nc_loss(zt_hbm, knnt_hbm, sidx_hbm, out_hbm,
                zring, kring, cen_v, nids_v, allids_v, cen_all_v, slots_v,
                cnt_v, acc_v, sidx_v, marked_s, spm_ids,
                spm_cen, sem_z, sem_k, sem_m):
        sc = lax.axis_index("c")
        tile = lax.axis_index("s")
        wid = sc * 16 + tile
        iota = lax.iota(jnp.int32, 16)
        zeros_i = jnp.zeros((16,), jnp.int32)
        ones_i = jnp.ones((16,), jnp.int32)

        pltpu.async_copy(
            sidx_hbm.at[pl.ds(wid * 32, 32)], sidx_v, sem_m).wait()
        svec = [sidx_v[pl.ds(0, 16)], sidx_v[pl.ds(16, 16)]]

        def dyn_splat(vec, idx16):
            return lax.gather(
                vec, _splat(idx16)[:, None],
                lax.GatherDimensionNumbers(
                    offset_dims=(), collapsed_slice_dims=(0,),
                    start_index_map=(0,)),
                slice_sizes=(1,),
                mode=lax.GatherScatterMode.PROMISE_IN_BOUNDS)

        def cnt_at(lc):
            vec = cnt_v[pl.ds((lc // 16) * 16, 16)]
            return dyn_splat(vec, lc % 16)[0]

        cal, _ = plsc.scan_count(zeros_i)
        rank_base = cal - iota

        for v in range(32):
            cnt_v[pl.ds(v * 16, 16)] = zeros_i
        acc_v[...] = jnp.zeros((16,), jnp.float32)

        def issue_a(i):
            cid = svec[i // 16][i % 16]
            c128 = lax.shift_right_logical(cid, 7) * 128
            pltpu.async_copy(
                zt_hbm.at[pl.ds(0, 32), pl.ds(c128, 128)],
                zring.at[i % 8], sem_z)
            pltpu.async_copy(
                knnt_hbm.at[pl.ds(0, 16), pl.ds(c128, 128)],
                kring.at[i % 8], sem_k)

        for i in range(8):
            issue_a(i)
        for i in range(32):
            s = i % 8
            pltpu.make_async_copy(
                zt_hbm.at[pl.ds(0, 32), pl.ds(0, 128)],
                zring.at[s], sem_z).wait()
            pltpu.make_async_copy(
                knnt_hbm.at[pl.ds(0, 16), pl.ds(0, 128)],
                kring.at[s], sem_k).wait()
            cid = svec[i // 16][i % 16]
            lane = jnp.bitwise_and(_splat(cid), 127)
            c_lo = plsc.load_gather(zring, [_splat(s), iota, lane])
            c_hi = plsc.load_gather(zring, [_splat(s), iota + 16, lane])
            cen_v[i // 4, pl.ds((i % 4) * 32, 16)] = c_lo
            cen_v[i // 4, pl.ds((i % 4) * 32 + 16, 16)] = c_hi
            nid = plsc.load_gather(kring, [_splat(s), iota, lane])
            valid = (wid * 32 + i) < _S
            nids_v[i // 8, pl.ds((i % 8) * 16, 16)] = jnp.where(
                valid, nid, _splat(_SENTINEL))
            if i + 8 < 32:
                issue_a(i + 8)

        pltpu.sync_copy(nids_v, spm_ids.at[pl.ds(tile * 4, 4), :])
        pltpu.sync_copy(cen_v, spm_cen.at[pl.ds(tile * 8, 8), :])
        plsc.subcore_barrier()

        pltpu.sync_copy(spm_ids, allids_v)
        pltpu.sync_copy(spm_cen, cen_all_v)

        def fbody2(r2, _):
            for u in range(2):
                fbody(r2 * 2 + u)
            return 0

        def fbody(r):
            ids = allids_v[r // 8, pl.ds((r % 8) * 16, 16)]
            cols = lax.shift_right_logical(ids, 7)
            mine = jnp.logical_and(
                jnp.bitwise_and(cols, 15) == tile, cols < _CT)
            lc = lax.shift_right_logical(cols, 4)
            pk = jnp.bitwise_or(
                lax.shift_left(r * 16 + iota, 7),
                jnp.bitwise_and(ids, 127))
            base = plsc.load_gather(cnt_v, [lc], mask=mine)
            rank, _ = plsc.scan_count(lc, mine)
            rank = rank - rank_base
            pos = lc * _CAP + jnp.minimum(base + rank, _CAP - 1)
            plsc.store_scatter(slots_v, [pos], pk, mask=mine)
            plsc.addupdate_scatter(cnt_v, [lc], ones_i, mask=mine)

        lax.fori_loop(0, 256, fbody2, 0)

        def mbody(lc, cur):
            cnt = cnt_at(lc)

            @pl.when(cnt > 0)
            def _():
                marked_s[cur] = lc

            return cur + jnp.where(cnt > 0, 1, 0)

        nmark = lax.fori_loop(0, _LCOLS, mbody, jnp.int32(0))

        def issue_c(m):
            lc = marked_s[m]
            c128 = (lc * 16 + tile) * 128
            for t in range(4):
                pltpu.async_copy(
                    zt_hbm.at[pl.ds(t * 8, 8), pl.ds(c128, 128)],
                    zring.at[m % 12, pl.ds(t * 8, 8), :], sem_z)

        for i in range(12):
            @pl.when(i < nmark)
            def _(i=i):
                issue_c(i)

        def cbody(m, _):
            s = m % 12
            for t in range(4):
                pltpu.make_async_copy(
                    zt_hbm.at[pl.ds(t * 8, 8), pl.ds(0, 128)],
                    zring.at[s, pl.ds(t * 8, 8), :], sem_z).wait()

            @pl.when(m + 12 < nmark)
            def _():
                issue_c(m + 12)

            lc = marked_s[m]
            cnt = jnp.minimum(cnt_at(lc), _CAP)

            pk0 = slots_v[pl.ds(lc * _CAP, 16)]
            pk1 = slots_v[pl.ds(lc * _CAP + 16, 16)]

            def match(j, pkvec, _):
                pkj = lax.gather(
                    pkvec, _splat(j)[:, None],
                    lax.GatherDimensionNumbers(
                        offset_dims=(), collapsed_slice_dims=(0,),
                        start_index_map=(0,)),
                    slice_sizes=(1,),
                    mode=lax.GatherScatterMode.PROMISE_IN_BOUNDS)
                lane = jnp.bitwise_and(pkj, 127)
                cloc = lax.shift_right_logical(pkj, 11)
                z_lo = plsc.load_gather(zring, [_splat(s), iota, lane])
                z_hi = plsc.load_gather(zring, [_splat(s), iota + 16, lane])
                flat_lo = cloc * 32 + iota
                flat_hi = flat_lo + 16
                c_lo = plsc.load_gather(
                    cen_all_v,
                    [lax.shift_right_logical(flat_lo, 7),
                     jnp.bitwise_and(flat_lo, 127)])
                c_hi = plsc.load_gather(
                    cen_all_v,
                    [lax.shift_right_logical(flat_hi, 7),
                     jnp.bitwise_and(flat_hi, 127)])
                num = jnp.sum(c_lo * z_lo + c_hi * z_hi)
                nn = jnp.sum(z_lo * z_lo + z_hi * z_hi)
                cn = jnp.sum(c_lo * c_lo + c_hi * c_hi)
                denom = jnp.maximum(nn, _EPS2) * jnp.maximum(cn, _EPS2)
                dv = jnp.full((16,), denom, jnp.float32)
                cos = jnp.full((16,), num, jnp.float32) * _rsqrt(dv)
                acc_v[...] = acc_v[...] + cos
                return 0

            lax.fori_loop(0, jnp.minimum(cnt, 16),
                          lambda j, c: match(j, pk0, c), 0)

            @pl.when(cnt > 16)
            def _():
                lax.fori_loop(0, cnt - 16,
                              lambda j, c: match(j, pk1, c), 0)

            return 0

        lax.fori_loop(0, nmark, cbody, 0)

        pltpu.sync_copy(acc_v, out_hbm.at[wid])

    return nc_loss


def kernel(z, knn_neighbors):
    n = z.shape[0]
    sample_size = min(1000, n)
    skey = jax.random.key(42)
    sample_indices = jax.random.randint(
        skey, (sample_size,), 0, n, dtype=jnp.int32)
    sidx = jnp.zeros((_SPAD,), jnp.int32).at[:sample_size].set(sample_indices)
    partials = _make_sc_kernel()(z.T, knn_neighbors.T, sidx)
    total_cos = jnp.sum(partials) / 16.0
    return 1.0 - total_cos / jnp.float32(sample_size * _K)


# speedup vs baseline: 1.0108x; 1.0108x over previous
import functools

import jax
import jax.numpy as jnp
from jax import lax
from jax.experimental import pallas as pl
from jax.experimental.pallas import tpu as pltpu
from jax.experimental.pallas import tpu_sc as plsc

_N = 1000000
_D = 32
_K = 16
_S = 1000
_SPAD = 1024
_CT = (_N + 127) // 128
_LCOLS = (_CT + 15) // 16
_CAP = 32
_SENTINEL = 0x0FFFFFFF
_EPS2 = 1e-16


def _rsqrt(x):
    xi = plsc.bitcast(x, jnp.int32)
    yi = jnp.int32(0x5F3759DF) - lax.shift_right_arithmetic(xi, 1)
    y = plsc.bitcast(yi, jnp.float32)
    for _ in range(3):
        y = y * (1.5 - 0.5 * x * y * y)
    return y


def _splat(x):
    return jnp.full((16,), x, jnp.int32)


def _make_sc_kernel():
    mesh = plsc.VectorSubcoreMesh(core_axis_name="c", subcore_axis_name="s")

    @functools.partial(
        pl.kernel,
        out_type=jax.ShapeDtypeStruct((32, 16), jnp.float32),
        mesh=mesh,
        compiler_params=pltpu.CompilerParams(
            needs_layout_passes=False, use_tc_tiling_on_sc=True),
        scratch_types=[
            pltpu.VMEM((12, 32, 128), jnp.float32),
            pltpu.VMEM((8, 16, 128), jnp.int32),
            pltpu.VMEM((8, 128), jnp.float32),
            pltpu.VMEM((4, 128), jnp.int32),
            pltpu.VMEM((64, 128), jnp.int32),
            pltpu.VMEM((128, 128), jnp.float32),
            pltpu.VMEM((_LCOLS * _CAP + 32,), jnp.int32),
            pltpu.VMEM((512,), jnp.int32),
            pltpu.VMEM((16,), jnp.float32),
            pltpu.VMEM((32,), jnp.int32),
            pltpu.SMEM((512,), jnp.int32),
            pltpu.VMEM_SHARED((64, 128), jnp.int32),
            pltpu.VMEM_SHARED((128, 128), jnp.float32),
            pltpu.SemaphoreType.DMA,
            pltpu.SemaphoreType.DMA,
            pltpu.SemaphoreType.DMA,
        ],
    )
    def nc_loss(zt_hbm, knnt_hbm, sidx_hbm, out_hbm,
                zring, kring, cen_v, nids_v, allids_v, cen_all_v, slots_v,
                cnt_v, acc_v, sidx_v, marked_s, spm_ids,
                spm_cen, sem_z, sem_k, sem_m):
        sc = lax.axis_index("c")
        tile = lax.axis_index("s")
        wid = sc * 16 + tile
        iota = lax.iota(jnp.int32, 16)
        zeros_i = jnp.zeros((16,), jnp.int32)
        ones_i = jnp.ones((16,), jnp.int32)

        pltpu.async_copy(
            sidx_hbm.at[pl.ds(wid * 32, 32)], sidx_v, sem_m).wait()
        svec = [sidx_v[pl.ds(0, 16)], sidx_v[pl.ds(16, 16)]]

        def dyn_splat(vec, idx16):
            return lax.gather(
                vec, _splat(idx16)[:, None],
                lax.GatherDimensionNumbers(
                    offset_dims=(), collapsed_slice_dims=(0,),
                    start_index_map=(0,)),
                slice_sizes=(1,),
                mode=lax.GatherScatterMode.PROMISE_IN_BOUNDS)

        def cnt_at(lc):
            vec = cnt_v[pl.ds((lc // 16) * 16, 16)]
            return dyn_splat(vec, lc % 16)[0]

        cal, _ = plsc.scan_count(zeros_i)
        rank_base = cal - iota

        for v in range(32):
            cnt_v[pl.ds(v * 16, 16)] = zeros_i
        acc_v[...] = jnp.zeros((16,), jnp.float32)

        def issue_a(i):
            cid = svec[i // 16][i % 16]
            c128 = lax.shift_right_logical(cid, 7) * 128
            pltpu.async_copy(
                zt_hbm.at[pl.ds(0, 32), pl.ds(c128, 128)],
                zring.at[i % 8], sem_z)
            pltpu.async_copy(
                knnt_hbm.at[pl.ds(0, 16), pl.ds(c128, 128)],
                kring.at[i % 8], sem_k)

        for i in range(8):
            issue_a(i)
        for i in range(32):
            s = i % 8
            pltpu.make_async_copy(
                zt_hbm.at[pl.ds(0, 32), pl.ds(0, 128)],
                zring.at[s], sem_z).wait()
            pltpu.make_async_copy(
                knnt_hbm.at[pl.ds(0, 16), pl.ds(0, 128)],
                kring.at[s], sem_k).wait()
            cid = svec[i // 16][i % 16]
            lane = jnp.bitwise_and(_splat(cid), 127)
            c_lo = plsc.load_gather(zring, [_splat(s), iota, lane])
            c_hi = plsc.load_gather(zring, [_splat(s), iota + 16, lane])
            cen_v[i // 4, pl.ds((i % 4) * 32, 16)] = c_lo
            cen_v[i // 4, pl.ds((i % 4) * 32 + 16, 16)] = c_hi
            nid = plsc.load_gather(kring, [_splat(s), iota, lane])
            valid = (wid * 32 + i) < _S
            nids_v[i // 8, pl.ds((i % 8) * 16, 16)] = jnp.where(
                valid, nid, _splat(_SENTINEL))
            if i + 8 < 32:
                issue_a(i + 8)

        pltpu.sync_copy(nids_v, spm_ids.at[pl.ds(tile * 4, 4), :])
        pltpu.sync_copy(cen_v, spm_cen.at[pl.ds(tile * 8, 8), :])
        plsc.subcore_barrier()

        pltpu.sync_copy(spm_ids, allids_v)
        pltpu.sync_copy(spm_cen, cen_all_v)

        def fbody2(r2, _):
            for u in range(2):
                fbody(r2 * 2 + u)
            return 0

        def fbody(r):
            ids = allids_v[r // 8, pl.ds((r % 8) * 16, 16)]
            cols = lax.shift_right_logical(ids, 7)
            mine = jnp.logical_and(
                jnp.bitwise_and(cols, 15) == tile, cols < _CT)
            lc = lax.shift_right_logical(cols, 4)
            pk = jnp.bitwise_or(
                lax.shift_left(r * 16 + iota, 7),
                jnp.bitwise_and(ids, 127))
            base = plsc.load_gather(cnt_v, [lc], mask=mine)
            rank, _ = plsc.scan_count(lc, mine)
            rank = rank - rank_base
            pos = lc * _CAP + jnp.minimum(base + rank, _CAP - 1)
            plsc.store_scatter(slots_v, [pos], pk, mask=mine)
            plsc.addupdate_scatter(cnt_v, [lc], ones_i, mask=mine)

        lax.fori_loop(0, 256, fbody2, 0)

        def mbody(lc, cur):
            cnt = cnt_at(lc)

            @pl.when(cnt > 0)
            def _():
                marked_s[cur] = lc

            return cur + jnp.where(cnt > 0, 1, 0)

        nmark = lax.fori_loop(0, _LCOLS, mbody, jnp.int32(0))

        def issue_c(m):
            lc = marked_s[m]
            c128 = (lc * 16 + tile) * 128
            pltpu.async_copy(
                zt_hbm.at[pl.ds(0, 32), pl.ds(c128, 128)],
                zring.at[m % 12], sem_z)

        for i in range(12):
            @pl.when(i < nmark)
            def _(i=i):
                issue_c(i)

        def cbody(m, _):
            s = m % 12
            pltpu.make_async_copy(
                zt_hbm.at[pl.ds(0, 32), pl.ds(0, 128)],
                zring.at[s], sem_z).wait()

            @pl.when(m + 12 < nmark)
            def _():
                issue_c(m + 12)

            lc = marked_s[m]
            cnt = jnp.minimum(cnt_at(lc), _CAP)

            pk0 = slots_v[pl.ds(lc * _CAP, 16)]
            pk1 = slots_v[pl.ds(lc * _CAP + 16, 16)]

            def match(j, pkvec, _):
                pkj = lax.gather(
                    pkvec, _splat(j)[:, None],
                    lax.GatherDimensionNumbers(
                        offset_dims=(), collapsed_slice_dims=(0,),
                        start_index_map=(0,)),
                    slice_sizes=(1,),
                    mode=lax.GatherScatterMode.PROMISE_IN_BOUNDS)
                lane = jnp.bitwise_and(pkj, 127)
                cloc = lax.shift_right_logical(pkj, 11)
                z_lo = plsc.load_gather(zring, [_splat(s), iota, lane])
                z_hi = plsc.load_gather(zring, [_splat(s), iota + 16, lane])
                flat_lo = cloc * 32 + iota
                flat_hi = flat_lo + 16
                c_lo = plsc.load_gather(
                    cen_all_v,
                    [lax.shift_right_logical(flat_lo, 7),
                     jnp.bitwise_and(flat_lo, 127)])
                c_hi = plsc.load_gather(
                    cen_all_v,
                    [lax.shift_right_logical(flat_hi, 7),
                     jnp.bitwise_and(flat_hi, 127)])
                num = jnp.sum(c_lo * z_lo + c_hi * z_hi)
                nn = jnp.sum(z_lo * z_lo + z_hi * z_hi)
                cn = jnp.sum(c_lo * c_lo + c_hi * c_hi)
                denom = jnp.maximum(nn, _EPS2) * jnp.maximum(cn, _EPS2)
                dv = jnp.full((16,), denom, jnp.float32)
                cos = jnp.full((16,), num, jnp.float32) * _rsqrt(dv)
                acc_v[...] = acc_v[...] + cos
                return 0

            lax.fori_loop(0, jnp.minimum(cnt, 16),
                          lambda j, c: match(j, pk0, c), 0)

            @pl.when(cnt > 16)
            def _():
                lax.fori_loop(0, cnt - 16,
                              lambda j, c: match(j, pk1, c), 0)

            return 0

        lax.fori_loop(0, nmark, cbody, 0)

        pltpu.sync_copy(acc_v, out_hbm.at[wid])

    return nc_loss


def kernel(z, knn_neighbors):
    n = z.shape[0]
    sample_size = min(1000, n)
    skey = jax.random.key(42)
    sample_indices = jax.random.randint(
        skey, (sample_size,), 0, n, dtype=jnp.int32)
    sidx = jnp.zeros((_SPAD,), jnp.int32).at[:sample_size].set(sample_indices)
    partials = _make_sc_kernel()(z.T, knn_neighbors.T, sidx)
    total_cos = jnp.sum(partials) / 16.0
    return 1.0 - total_cos / jnp.float32(sample_size * _K)
